# baseline (speedup 1.0000x reference)
import functools

import jax
import jax.numpy as jnp
from jax import lax
from jax.experimental import pallas as pl
from jax.experimental.pallas import tpu as pltpu
from jax.experimental.pallas import tpu_sc as plsc

_NC = 2
_NS = 16
_NW = _NC * _NS
_L = 16

_B = 16
_R = 512
_W = 512
_RPW = _B * _R // _NW
_ROWS = 32
_NCHUNK = 0
_GPR = _W // (8 * _L)

_P1 = 0.9992354838332709
_P2 = -0.49023072342338675
_P3 = 0.2852726810905121
_P4 = -0.13158182508868854
_P5 = 0.030449004538639933


def _sc_body(pred_hbm, targ_hbm, out_hbm,
             pb0, pb1, tb0, tb1, acc, sp0, sp1, st0, st1):
    wid = lax.axis_index("s") * _NC + lax.axis_index("c")
    b = wid // 2
    r0 = (wid % 2) * _RPW

    pbufs = (pb0, pb1)
    tbufs = (tb0, tb1)
    psems = (sp0, sp1)
    tsems = (st0, st1)
    hp = [None, None]
    ht = [None, None]

    def issue(c):
        k = c % 2
        rows = r0 + c * _ROWS
        hp[k] = pltpu.async_copy(
            pred_hbm.at[b, 0, pl.ds(rows, _ROWS), :], pbufs[k], psems[k])
        ht[k] = pltpu.async_copy(
            targ_hbm.at[b, 0, pl.ds(rows, _ROWS), :], tbufs[k], tsems[k])

    def chunk_sum(pb, tb, carry):
        def row_step(r, carry):
            def grp_step(g, carry):
                ls, cs = carry
                cb = g * (8 * _L)
                ls2 = list(ls)
                cs2 = list(cs)
                for k in range(8):
                    p = pb[r, pl.ds(cb + k * _L, _L)]
                    t = tb[r, pl.ds(cb + k * _L, _L)]
                    npv = -p
                    u = jnp.exp(jnp.minimum(p, npv))
                    l = u * (_P1 + u * (_P2 + u * (_P3 + u * (_P4 + u * _P5))))
                    q = jnp.where(t > 0, npv, p)
                    m = jnp.where(t != 1, 1.0, 0.0)
                    per = jnp.maximum(q, 0.0) + l
                    j = k % 4
                    ls2[j] = ls2[j] + per * m
                    cs2[j] = cs2[j] + m
                return tuple(ls2), tuple(cs2)

            return lax.fori_loop(0, _GPR, grp_step, carry)

        return lax.fori_loop(0, _ROWS, row_step, carry)

    if _NCHUNK:
        issue(0)
    z = jnp.zeros((_L,), jnp.float32)
    carry = ((z,) * 4, (z,) * 4)
    for c in range(_NCHUNK):
        if c + 1 < _NCHUNK:
            issue(c + 1)
        hp[c % 2].wait()
        ht[c % 2].wait()
        carry = chunk_sum(pbufs[c % 2], tbufs[c % 2], carry)

    ls, cs = carry
    lt = (ls[0] + ls[1]) + (ls[2] + ls[3])
    ct = (cs[0] + cs[1]) + (cs[2] + cs[3])
    acc[pl.ds(0, _L)] = lt
    acc[pl.ds(_L, _L)] = ct
    pltpu.sync_copy(acc, out_hbm.at[wid])


@functools.cache
def _sc_partials():
    return pl.kernel(
        _sc_body,
        out_type=jax.ShapeDtypeStruct((_NW, 2 * _L), jnp.float32),
        mesh=plsc.VectorSubcoreMesh(core_axis_name="c", subcore_axis_name="s"),
        scratch_types=[
            pltpu.VMEM((_ROWS, _W), jnp.float32),
            pltpu.VMEM((_ROWS, _W), jnp.float32),
            pltpu.VMEM((_ROWS, _W), jnp.int32),
            pltpu.VMEM((_ROWS, _W), jnp.int32),
            pltpu.VMEM((2 * _L,), jnp.float32),
            pltpu.SemaphoreType.DMA,
            pltpu.SemaphoreType.DMA,
            pltpu.SemaphoreType.DMA,
            pltpu.SemaphoreType.DMA,
        ],
        compiler_params=pltpu.CompilerParams(
            use_tc_tiling_on_sc=True, skip_device_barrier=True),
    )


_BTC = 16


def _tc_body(p_ref, t_ref, o_ref):
    i = pl.program_id(0)
    p = p_ref[0, 0, :, :]
    t = t_ref[0, 0, :, :]
    m = (t != 1).astype(jnp.float32)
    z = (t > 0).astype(jnp.float32)
    per = jnp.maximum(p, 0.0) - p * z + jnp.log1p(jnp.exp(-jnp.abs(p)))
    s = jnp.sum(per * m)
    c = jnp.sum(m)

    @pl.when(i == 0)
    def _init():
        o_ref[0, 0] = s
        o_ref[0, 1] = c

    @pl.when(i > 0)
    def _accum():
        o_ref[0, 0] += s
        o_ref[0, 1] += c


_tc_worker = pl.pallas_call(
    _tc_body,
    grid=(_BTC,),
    in_specs=[
        pl.BlockSpec((1, 1, _R, _W), lambda i: (_B - _BTC + i, 0, 0, 0)),
        pl.BlockSpec((1, 1, _R, _W), lambda i: (_B - _BTC + i, 0, 0, 0)),
    ],
    out_specs=pl.BlockSpec(memory_space=pltpu.SMEM),
    out_shape=jax.ShapeDtypeStruct((1, 2), jnp.float32),
    compiler_params=pltpu.CompilerParams(
        dimension_semantics=("arbitrary",)),
)


def _finish_body(acc_ref, tcp_ref, out_ref):
    s = jnp.sum(acc_ref[:, 0:_L]) + tcp_ref[0, 0]
    c = jnp.sum(acc_ref[:, _L:2 * _L]) + tcp_ref[0, 1]
    out_ref[0, 0] = s / c


_finish = pl.pallas_call(
    _finish_body,
    in_specs=[
        pl.BlockSpec((_NW, 2 * _L), lambda: (0, 0)),
        pl.BlockSpec(memory_space=pltpu.SMEM),
    ],
    out_shape=jax.ShapeDtypeStruct((1, 1), jnp.float32),
    out_specs=pl.BlockSpec(memory_space=pltpu.SMEM),
)


def kernel(pred, target):
    partials = _sc_partials()(pred, target)
    tcp = _tc_worker(pred, target)
    return _finish(partials, tcp).reshape(())
